# baseline (speedup 1.0000x reference)
import jax
import jax.numpy as jnp
from jax import lax
from jax.experimental import pallas as pl
from jax.experimental.pallas import tpu as pltpu
from jax.experimental.pallas import tpu_sc as plsc

Q, L, D, K, TOPK = 1024, 10, 512, 100000, 10
NCAND = 2 * TOPK
KBLK = 2048
NKB = (K + KBLK - 1) // KBLK
KPAD = NKB * KBLK
GRP = 128
NGRP = KPAD // GRP
NGRP_M = 896
NEG = -1e30
BIGI = 2**30
BQ = 128
NSEL = 32

NC, NS, LN = 2, 16, 16
NW = NC * NS
QPW = Q // NW


def _ln(x, g, b):
    m = jnp.mean(x, axis=-1, keepdims=True)
    v = jnp.mean((x - m) ** 2, axis=-1, keepdims=True)
    return (x - m) / jnp.sqrt(v + 1e-5) * g + b


def _enc_kernel(ctx_ref, q_ref, wctx_ref, w1_ref, b1_ref, lng_ref, lnb_ref,
                w2_ref, b2_ref, eq_ref, cv_ref):
    qb = q_ref[...]
    acc = None
    cv = None
    for l in range(L):
        x = ctx_ref[:, l * D:(l + 1) * D]
        cv = x if cv is None else cv + x
        p = jax.nn.gelu(jnp.dot(x, wctx_ref[...], preferred_element_type=jnp.float32))
        comb = jnp.concatenate([p, qb], axis=1)
        h = jnp.dot(comb, w1_ref[...], preferred_element_type=jnp.float32) + b1_ref[...]
        h = jax.nn.gelu(_ln(h, lng_ref[...], lnb_ref[...]))
        caq = jnp.dot(h, w2_ref[...], preferred_element_type=jnp.float32) + b2_ref[...]
        acc = caq if acc is None else acc + caq
    eq_ref[...] = acc / float(L)
    cv_ref[...] = cv / float(L)


def _score_kernel(eq_ref, keys_ref, s_ref, m_ref):
    kb = pl.program_id(0)
    s = lax.dot_general(eq_ref[...], keys_ref[...], (((1,), (1,)), ((), ())),
                        preferred_element_type=jnp.float32)
    col = kb * KBLK + lax.broadcasted_iota(jnp.int32, (Q, KBLK), 1)
    s = jnp.where(col < K, s, NEG)
    s_ref[...] = s
    mloc = jnp.max(s.reshape(Q, KBLK // GRP, GRP), axis=2)
    sub = kb % 8
    NGB = KBLK // GRP

    @pl.when(sub == 0)
    def _():
        m_ref[...] = jnp.full((Q, 8 * NGB), NEG, jnp.float32)
        m_ref[:, 0:NGB] = mloc

    for _t in range(1, 8):
        @pl.when(sub == _t)
        def _(t=_t):
            m_ref[:, t * NGB:(t + 1) * NGB] = mloc


def _gsel_kernel(gmax_ref, gsel_ref):
    cur = gmax_ref[...]
    gpos = lax.broadcasted_iota(jnp.int32, (BQ, NGRP_M), 1)
    g0 = None
    for j in range(NCAND):
        m = jnp.max(cur, axis=1, keepdims=True)
        g = jnp.min(jnp.where(cur == m, gpos, BIGI), axis=1, keepdims=True)
        gsel_ref[:, j:j + 1] = g
        cur = jnp.where(gpos == g, NEG, cur)
        if j == 0:
            g0 = g
    for j in range(NCAND, NSEL):
        gsel_ref[:, j:j + 1] = g0


def _sgath_kernel(srows_hbm, gsel_hbm, cg_hbm, idv, chunk, sem):
    wid = lax.axis_index("s") * NC + lax.axis_index("c")

    def per_query(j, _c):
        q = wid * QPW + j
        pltpu.sync_copy(gsel_hbm.at[pl.ds(q * NSEL, NSEL)], idv)
        base = jnp.full((LN,), q * NGRP, jnp.int32)
        iv0 = idv[pl.ds(0, LN)] + base
        iv1 = idv[pl.ds(LN, LN)] + base
        d0 = pltpu.async_copy(srows_hbm.at[iv0], chunk.at[pl.ds(0, LN)], sem)
        d1 = pltpu.async_copy(srows_hbm.at[iv1], chunk.at[pl.ds(LN, LN)], sem)
        d0.wait()
        d1.wait()
        pltpu.sync_copy(chunk, cg_hbm.at[pl.ds(q * NSEL, NSEL)])
        return 0

    lax.fori_loop(0, QPW, per_query, 0)


def _tsel_kernel(cg_ref, gsel_ref, cidx_ref, gidx_scr):
    io128 = lax.broadcasted_iota(jnp.int32, (BQ, GRP), 1)
    for r in range(NSEL):
        gidx_scr[:, r * GRP:(r + 1) * GRP] = \
            gsel_ref[:, r:r + 1] * GRP + io128
    gidx = gidx_scr[...]
    lane = lax.broadcasted_iota(jnp.int32, (BQ, NSEL * GRP), 1)
    vals = jnp.where(lane < NCAND * GRP, cg_ref[...], NEG)
    c0 = None
    for j in range(NCAND):
        m = jnp.max(vals, axis=1, keepdims=True)
        sel = jnp.min(jnp.where(vals == m, gidx, BIGI), axis=1, keepdims=True)
        cidx_ref[:, j:j + 1] = sel
        vals = jnp.where(gidx == sel, NEG, vals)
        if j == 0:
            c0 = sel
    for j in range(NCAND, NSEL):
        cidx_ref[:, j:j + 1] = c0


def _kgath_kernel(keys_hbm, cidx_hbm, cemb_hbm, idv, krows, sem):
    wid = lax.axis_index("s") * NC + lax.axis_index("c")

    def per_query(j, _c):
        q = wid * QPW + j
        pltpu.sync_copy(cidx_hbm.at[pl.ds(q * NSEL, NSEL)], idv)
        iv0 = idv[pl.ds(0, LN)]
        iv1 = idv[pl.ds(LN, LN)]
        d0 = pltpu.async_copy(keys_hbm.at[iv0], krows.at[pl.ds(0, LN)], sem)
        d1 = pltpu.async_copy(keys_hbm.at[iv1], krows.at[pl.ds(LN, LN)], sem)
        d0.wait()
        d1.wait()
        pltpu.sync_copy(krows.at[pl.ds(0, 24)], cemb_hbm.at[pl.ds(q * 24, 24)])
        return 0

    lax.fori_loop(0, QPW, per_query, 0)


def _rr_kernel(cemb_ref, q_ref, cv_ref, w1_ref, b1_ref, lng_ref, lnb_ref,
               w2_ref, b2_ref, cidT_ref, fs_ref, fi_ref, rel_scr):
    qb = q_ref[...]
    cvb = cv_ref[...]
    z = cemb_ref[...].reshape(BQ, 24, D)
    for c in range(NCAND):
        x = z[:, c, :]
        inp = jnp.concatenate([x, qb, cvb], axis=1)
        h = jnp.dot(inp, w1_ref[...], preferred_element_type=jnp.float32) + b1_ref[...]
        h = jax.nn.gelu(_ln(h, lng_ref[...], lnb_ref[...]))
        r = jnp.dot(h, w2_ref[...], preferred_element_type=jnp.float32) + b2_ref[...]
        rel_scr[c:c + 1, :] = jax.nn.sigmoid(r)[:, 0][None, :]

    relm = rel_scr[...]
    ci = cidT_ref[...]
    pos = lax.broadcasted_iota(jnp.int32, (NCAND, BQ), 0)
    cur = relm
    for j in range(TOPK):
        m = jnp.max(cur, axis=0, keepdims=True)
        pstar = jnp.min(jnp.where(cur == m, pos, NCAND), axis=0,
                        keepdims=True)
        sel = pos == pstar
        fs_ref[j:j + 1, :] = m
        fi_ref[j:j + 1, :] = jnp.sum(jnp.where(sel, ci, 0), axis=0,
                                     keepdims=True)
        cur = jnp.where(sel, -1.0, cur)


def kernel(query, context, keys, W_ctx, fusion_W1, fusion_b1, fusion_ln_g,
           fusion_ln_b, fusion_W2, fusion_b2, rr_W1, rr_b1, rr_ln_g, rr_ln_b,
           rr_W2, rr_b2, top_k):
    f32 = jnp.float32
    i32 = jnp.int32
    b1r = fusion_b1.reshape(1, D)
    lngr = fusion_ln_g.reshape(1, D)
    lnbr = fusion_ln_b.reshape(1, D)
    b2r = fusion_b2.reshape(1, D)
    rrb1r = rr_b1.reshape(1, D)
    rlngr = rr_ln_g.reshape(1, D)
    rlnbr = rr_ln_b.reshape(1, D)
    rb2r = rr_b2.reshape(1, 1)

    ctx2 = context.reshape(Q, L * D)

    eq, cvec = pl.pallas_call(
        _enc_kernel,
        grid=(Q // BQ,),
        in_specs=[
            pl.BlockSpec((BQ, L * D), lambda i: (i, 0)),
            pl.BlockSpec((BQ, D), lambda i: (i, 0)),
            pl.BlockSpec((D, D), lambda i: (0, 0)),
            pl.BlockSpec((2 * D, D), lambda i: (0, 0)),
            pl.BlockSpec((1, D), lambda i: (0, 0)),
            pl.BlockSpec((1, D), lambda i: (0, 0)),
            pl.BlockSpec((1, D), lambda i: (0, 0)),
            pl.BlockSpec((D, D), lambda i: (0, 0)),
            pl.BlockSpec((1, D), lambda i: (0, 0)),
        ],
        out_specs=[
            pl.BlockSpec((BQ, D), lambda i: (i, 0)),
            pl.BlockSpec((BQ, D), lambda i: (i, 0)),
        ],
        out_shape=[
            jax.ShapeDtypeStruct((Q, D), f32),
            jax.ShapeDtypeStruct((Q, D), f32),
        ],
    )(ctx2, query, W_ctx, fusion_W1, b1r, lngr, lnbr, fusion_W2, b2r)

    scores, gmax = pl.pallas_call(
        _score_kernel,
        grid=(NKB,),
        in_specs=[
            pl.BlockSpec((Q, D), lambda k: (0, 0)),
            pl.BlockSpec((KBLK, D), lambda k: (k, 0)),
        ],
        out_specs=[
            pl.BlockSpec((Q, KBLK), lambda k: (0, k)),
            pl.BlockSpec((Q, 128), lambda k: (0, k // 8)),
        ],
        out_shape=[
            jax.ShapeDtypeStruct((Q, KPAD), f32),
            jax.ShapeDtypeStruct((Q, NGRP_M), f32),
        ],
    )(eq, keys)

    gsel = pl.pallas_call(
        _gsel_kernel,
        grid=(Q // BQ,),
        in_specs=[pl.BlockSpec((BQ, NGRP_M), lambda i: (i, 0))],
        out_specs=pl.BlockSpec((BQ, NSEL), lambda i: (i, 0)),
        out_shape=jax.ShapeDtypeStruct((Q, NSEL), i32),
    )(gmax)

    srows = scores.reshape(Q * NGRP, GRP)
    gself = gsel.reshape(Q * NSEL)

    sgath = pl.kernel(
        _sgath_kernel,
        out_type=jax.ShapeDtypeStruct((Q * NSEL, GRP), f32),
        mesh=plsc.VectorSubcoreMesh(core_axis_name="c", subcore_axis_name="s"),
        scratch_types=[
            pltpu.VMEM((NSEL,), i32),
            pltpu.VMEM((NSEL, GRP), f32),
            pltpu.SemaphoreType.DMA,
        ],
    )
    cg = sgath(srows, gself)

    cidx = pl.pallas_call(
        _tsel_kernel,
        grid=(Q // BQ,),
        in_specs=[
            pl.BlockSpec((BQ, NSEL * GRP), lambda i: (i, 0)),
            pl.BlockSpec((BQ, NSEL), lambda i: (i, 0)),
        ],
        out_specs=pl.BlockSpec((BQ, NSEL), lambda i: (i, 0)),
        out_shape=jax.ShapeDtypeStruct((Q, NSEL), i32),
        scratch_shapes=[pltpu.VMEM((BQ, NSEL * GRP), i32)],
    )(cg.reshape(Q, NSEL * GRP), gsel)

    kgath = pl.kernel(
        _kgath_kernel,
        out_type=jax.ShapeDtypeStruct((Q * 24, D), f32),
        mesh=plsc.VectorSubcoreMesh(core_axis_name="c", subcore_axis_name="s"),
        scratch_types=[
            pltpu.VMEM((NSEL,), i32),
            pltpu.VMEM((NSEL, D), f32),
            pltpu.SemaphoreType.DMA,
        ],
    )
    cemb = kgath(keys, cidx.reshape(Q * NSEL))

    cidT = cidx[:, :NCAND].T
    cemb2 = cemb.reshape(Q, 24 * D)

    fsT, fiT = pl.pallas_call(
        _rr_kernel,
        grid=(Q // BQ,),
        in_specs=[
            pl.BlockSpec((BQ, 24 * D), lambda i: (i, 0)),
            pl.BlockSpec((BQ, D), lambda i: (i, 0)),
            pl.BlockSpec((BQ, D), lambda i: (i, 0)),
            pl.BlockSpec((3 * D, D), lambda i: (0, 0)),
            pl.BlockSpec((1, D), lambda i: (0, 0)),
            pl.BlockSpec((1, D), lambda i: (0, 0)),
            pl.BlockSpec((1, D), lambda i: (0, 0)),
            pl.BlockSpec((D, 1), lambda i: (0, 0)),
            pl.BlockSpec((1, 1), lambda i: (0, 0)),
            pl.BlockSpec((NCAND, BQ), lambda i: (0, i)),
        ],
        out_specs=[
            pl.BlockSpec((TOPK, BQ), lambda i: (0, i)),
            pl.BlockSpec((TOPK, BQ), lambda i: (0, i)),
        ],
        out_shape=[
            jax.ShapeDtypeStruct((TOPK, Q), f32),
            jax.ShapeDtypeStruct((TOPK, Q), jnp.int32),
        ],
        scratch_shapes=[pltpu.VMEM((NCAND, BQ), f32)],
    )(cemb2, query, cvec, rr_W1, rrb1r, rlngr, rlnbr, rr_W2, rb2r, cidT)

    return fsT.T, fiT.T


# speedup vs baseline: 1.0282x; 1.0282x over previous
import jax
import jax.numpy as jnp
from jax import lax
from jax.experimental import pallas as pl
from jax.experimental.pallas import tpu as pltpu
from jax.experimental.pallas import tpu_sc as plsc

Q, L, D, K, TOPK = 1024, 10, 512, 100000, 10
NCAND = 2 * TOPK
KBLK = 2048
NKB = (K + KBLK - 1) // KBLK
KPAD = NKB * KBLK
GRP = 128
NGRP = KPAD // GRP
NGRP_M = 896
NEG = -1e30
BIGI = 2**30
BQ = 128
NSEL = 32

NC, NS, LN = 2, 16, 16
NW = NC * NS
QPW = Q // NW


def _ln(x, g, b):
    m = jnp.mean(x, axis=-1, keepdims=True)
    v = jnp.mean((x - m) ** 2, axis=-1, keepdims=True)
    return (x - m) / jnp.sqrt(v + 1e-5) * g + b


def _enc_kernel(ctx_ref, q_ref, wctx_ref, w1_ref, b1_ref, lng_ref, lnb_ref,
                w2_ref, b2_ref, eq_ref, cv_ref):
    qb = q_ref[...]
    acc = None
    cv = None
    for l in range(L):
        x = ctx_ref[:, l * D:(l + 1) * D]
        cv = x if cv is None else cv + x
        p = jax.nn.gelu(jnp.dot(x, wctx_ref[...], preferred_element_type=jnp.float32))
        comb = jnp.concatenate([p, qb], axis=1)
        h = jnp.dot(comb, w1_ref[...], preferred_element_type=jnp.float32) + b1_ref[...]
        h = jax.nn.gelu(_ln(h, lng_ref[...], lnb_ref[...]))
        caq = jnp.dot(h, w2_ref[...], preferred_element_type=jnp.float32) + b2_ref[...]
        acc = caq if acc is None else acc + caq
    eq_ref[...] = acc / float(L)
    cv_ref[...] = cv / float(L)


def _score_kernel(eq_ref, keys_ref, s_ref, m_ref):
    kb = pl.program_id(0)
    s = lax.dot_general(eq_ref[...], keys_ref[...], (((1,), (1,)), ((), ())),
                        preferred_element_type=jnp.float32)
    col = kb * KBLK + lax.broadcasted_iota(jnp.int32, (Q, KBLK), 1)
    s = jnp.where(col < K, s, NEG)
    s_ref[...] = s
    mloc = jnp.max(s.reshape(Q, KBLK // GRP, GRP), axis=2)
    sub = kb % 8
    NGB = KBLK // GRP

    @pl.when(sub == 0)
    def _():
        m_ref[...] = jnp.full((Q, 8 * NGB), NEG, jnp.float32)
        m_ref[:, 0:NGB] = mloc

    for _t in range(1, 8):
        @pl.when(sub == _t)
        def _(t=_t):
            m_ref[:, t * NGB:(t + 1) * NGB] = mloc


def _gsel_kernel(gmax_ref, gsel_ref):
    cur = gmax_ref[...]
    gpos = lax.broadcasted_iota(jnp.int32, (BQ, NGRP_M), 1)
    g0 = None
    for j in range(NCAND):
        m = jnp.max(cur, axis=1, keepdims=True)
        g = jnp.min(jnp.where(cur == m, gpos, BIGI), axis=1, keepdims=True)
        gsel_ref[:, j:j + 1] = g
        cur = jnp.where(gpos == g, NEG, cur)
        if j == 0:
            g0 = g
    for j in range(NCAND, NSEL):
        gsel_ref[:, j:j + 1] = g0


def _sgath_kernel(srows_hbm, gsel_hbm, cg_hbm, idv, chunk, sem):
    wid = lax.axis_index("s") * NC + lax.axis_index("c")
    q0 = wid * QPW
    pltpu.sync_copy(gsel_hbm.at[pl.ds(q0 * NSEL, QPW * NSEL)], idv)

    def issue(j):
        off = (j % 2) * NSEL
        base = jnp.full((LN,), (q0 + j) * NGRP, jnp.int32)
        iv0 = idv[pl.ds(j * NSEL, LN)] + base
        iv1 = idv[pl.ds(j * NSEL + LN, LN)] + base
        pltpu.async_copy(srows_hbm.at[iv0], chunk.at[pl.ds(off, LN)], sem)
        pltpu.async_copy(srows_hbm.at[iv1], chunk.at[pl.ds(off + LN, LN)], sem)

    issue(0)

    def per_query(j, _c):
        off = (j % 2) * NSEL
        pltpu.make_async_copy(srows_hbm.at[pl.ds(0, NSEL)],
                              chunk.at[pl.ds(off, NSEL)], sem).wait()

        @pl.when(j + 1 < QPW)
        def _():
            issue(j + 1)

        pltpu.sync_copy(chunk.at[pl.ds(off, NSEL)],
                        cg_hbm.at[pl.ds((q0 + j) * NSEL, NSEL)])
        return 0

    lax.fori_loop(0, QPW, per_query, 0)


def _tsel_kernel(cg_ref, gsel_ref, cidx_ref, gidx_scr):
    io128 = lax.broadcasted_iota(jnp.int32, (BQ, GRP), 1)
    for r in range(NSEL):
        gidx_scr[:, r * GRP:(r + 1) * GRP] = \
            gsel_ref[:, r:r + 1] * GRP + io128
    gidx = gidx_scr[...]
    lane = lax.broadcasted_iota(jnp.int32, (BQ, NSEL * GRP), 1)
    vals = jnp.where(lane < NCAND * GRP, cg_ref[...], NEG)
    c0 = None
    for j in range(NCAND):
        m = jnp.max(vals, axis=1, keepdims=True)
        sel = jnp.min(jnp.where(vals == m, gidx, BIGI), axis=1, keepdims=True)
        cidx_ref[:, j:j + 1] = sel
        vals = jnp.where(gidx == sel, NEG, vals)
        if j == 0:
            c0 = sel
    for j in range(NCAND, NSEL):
        cidx_ref[:, j:j + 1] = c0


def _kgath_kernel(keys_hbm, cidx_hbm, cemb_hbm, idv, krows, sem):
    wid = lax.axis_index("s") * NC + lax.axis_index("c")
    q0 = wid * QPW
    pltpu.sync_copy(cidx_hbm.at[pl.ds(q0 * NSEL, QPW * NSEL)], idv)

    def issue(j):
        off = (j % 2) * NSEL
        iv0 = idv[pl.ds(j * NSEL, LN)]
        iv1 = idv[pl.ds(j * NSEL + LN, LN)]
        pltpu.async_copy(keys_hbm.at[iv0], krows.at[pl.ds(off, LN)], sem)
        pltpu.async_copy(keys_hbm.at[iv1], krows.at[pl.ds(off + LN, LN)], sem)

    issue(0)

    def per_query(j, _c):
        off = (j % 2) * NSEL
        pltpu.make_async_copy(keys_hbm.at[pl.ds(0, NSEL)],
                              krows.at[pl.ds(off, NSEL)], sem).wait()

        @pl.when(j + 1 < QPW)
        def _():
            issue(j + 1)

        pltpu.sync_copy(krows.at[pl.ds(off, 24)],
                        cemb_hbm.at[pl.ds((q0 + j) * 24, 24)])
        return 0

    lax.fori_loop(0, QPW, per_query, 0)


def _rr_kernel(cemb_ref, q_ref, cv_ref, w1_ref, b1_ref, lng_ref, lnb_ref,
               w2_ref, b2_ref, cidT_ref, fs_ref, fi_ref, rel_scr):
    qb = q_ref[...]
    cvb = cv_ref[...]
    z = cemb_ref[...].reshape(BQ, 24, D)
    for c in range(NCAND):
        x = z[:, c, :]
        inp = jnp.concatenate([x, qb, cvb], axis=1)
        h = jnp.dot(inp, w1_ref[...], preferred_element_type=jnp.float32) + b1_ref[...]
        h = jax.nn.gelu(_ln(h, lng_ref[...], lnb_ref[...]))
        r = jnp.dot(h, w2_ref[...], preferred_element_type=jnp.float32) + b2_ref[...]
        rel_scr[c:c + 1, :] = jax.nn.sigmoid(r)[:, 0][None, :]

    relm = rel_scr[...]
    ci = cidT_ref[...]
    pos = lax.broadcasted_iota(jnp.int32, (NCAND, BQ), 0)
    cur = relm
    for j in range(TOPK):
        m = jnp.max(cur, axis=0, keepdims=True)
        pstar = jnp.min(jnp.where(cur == m, pos, NCAND), axis=0,
                        keepdims=True)
        sel = pos == pstar
        fs_ref[j:j + 1, :] = m
        fi_ref[j:j + 1, :] = jnp.sum(jnp.where(sel, ci, 0), axis=0,
                                     keepdims=True)
        cur = jnp.where(sel, -1.0, cur)


def kernel(query, context, keys, W_ctx, fusion_W1, fusion_b1, fusion_ln_g,
           fusion_ln_b, fusion_W2, fusion_b2, rr_W1, rr_b1, rr_ln_g, rr_ln_b,
           rr_W2, rr_b2, top_k):
    f32 = jnp.float32
    i32 = jnp.int32
    b1r = fusion_b1.reshape(1, D)
    lngr = fusion_ln_g.reshape(1, D)
    lnbr = fusion_ln_b.reshape(1, D)
    b2r = fusion_b2.reshape(1, D)
    rrb1r = rr_b1.reshape(1, D)
    rlngr = rr_ln_g.reshape(1, D)
    rlnbr = rr_ln_b.reshape(1, D)
    rb2r = rr_b2.reshape(1, 1)

    ctx2 = context.reshape(Q, L * D)

    eq, cvec = pl.pallas_call(
        _enc_kernel,
        grid=(Q // BQ,),
        in_specs=[
            pl.BlockSpec((BQ, L * D), lambda i: (i, 0)),
            pl.BlockSpec((BQ, D), lambda i: (i, 0)),
            pl.BlockSpec((D, D), lambda i: (0, 0)),
            pl.BlockSpec((2 * D, D), lambda i: (0, 0)),
            pl.BlockSpec((1, D), lambda i: (0, 0)),
            pl.BlockSpec((1, D), lambda i: (0, 0)),
            pl.BlockSpec((1, D), lambda i: (0, 0)),
            pl.BlockSpec((D, D), lambda i: (0, 0)),
            pl.BlockSpec((1, D), lambda i: (0, 0)),
        ],
        out_specs=[
            pl.BlockSpec((BQ, D), lambda i: (i, 0)),
            pl.BlockSpec((BQ, D), lambda i: (i, 0)),
        ],
        out_shape=[
            jax.ShapeDtypeStruct((Q, D), f32),
            jax.ShapeDtypeStruct((Q, D), f32),
        ],
    )(ctx2, query, W_ctx, fusion_W1, b1r, lngr, lnbr, fusion_W2, b2r)

    scores, gmax = pl.pallas_call(
        _score_kernel,
        grid=(NKB,),
        in_specs=[
            pl.BlockSpec((Q, D), lambda k: (0, 0)),
            pl.BlockSpec((KBLK, D), lambda k: (k, 0)),
        ],
        out_specs=[
            pl.BlockSpec((Q, KBLK), lambda k: (0, k)),
            pl.BlockSpec((Q, 128), lambda k: (0, k // 8)),
        ],
        out_shape=[
            jax.ShapeDtypeStruct((Q, KPAD), f32),
            jax.ShapeDtypeStruct((Q, NGRP_M), f32),
        ],
    )(eq, keys)

    gsel = pl.pallas_call(
        _gsel_kernel,
        grid=(Q // BQ,),
        in_specs=[pl.BlockSpec((BQ, NGRP_M), lambda i: (i, 0))],
        out_specs=pl.BlockSpec((BQ, NSEL), lambda i: (i, 0)),
        out_shape=jax.ShapeDtypeStruct((Q, NSEL), i32),
    )(gmax)

    srows = scores.reshape(Q * NGRP, GRP)
    gself = gsel.reshape(Q * NSEL)

    sgath = pl.kernel(
        _sgath_kernel,
        out_type=jax.ShapeDtypeStruct((Q * NSEL, GRP), f32),
        mesh=plsc.VectorSubcoreMesh(core_axis_name="c", subcore_axis_name="s"),
        scratch_types=[
            pltpu.VMEM((QPW * NSEL,), i32),
            pltpu.VMEM((2 * NSEL, GRP), f32),
            pltpu.SemaphoreType.DMA,
        ],
    )
    cg = sgath(srows, gself)

    cidx = pl.pallas_call(
        _tsel_kernel,
        grid=(Q // BQ,),
        in_specs=[
            pl.BlockSpec((BQ, NSEL * GRP), lambda i: (i, 0)),
            pl.BlockSpec((BQ, NSEL), lambda i: (i, 0)),
        ],
        out_specs=pl.BlockSpec((BQ, NSEL), lambda i: (i, 0)),
        out_shape=jax.ShapeDtypeStruct((Q, NSEL), i32),
        scratch_shapes=[pltpu.VMEM((BQ, NSEL * GRP), i32)],
    )(cg.reshape(Q, NSEL * GRP), gsel)

    kgath = pl.kernel(
        _kgath_kernel,
        out_type=jax.ShapeDtypeStruct((Q * 24, D), f32),
        mesh=plsc.VectorSubcoreMesh(core_axis_name="c", subcore_axis_name="s"),
        scratch_types=[
            pltpu.VMEM((QPW * NSEL,), i32),
            pltpu.VMEM((2 * NSEL, D), f32),
            pltpu.SemaphoreType.DMA,
        ],
    )
    cemb = kgath(keys, cidx.reshape(Q * NSEL))

    cidT = cidx[:, :NCAND].T
    cemb2 = cemb.reshape(Q, 24 * D)

    fsT, fiT = pl.pallas_call(
        _rr_kernel,
        grid=(Q // BQ,),
        in_specs=[
            pl.BlockSpec((BQ, 24 * D), lambda i: (i, 0)),
            pl.BlockSpec((BQ, D), lambda i: (i, 0)),
            pl.BlockSpec((BQ, D), lambda i: (i, 0)),
            pl.BlockSpec((3 * D, D), lambda i: (0, 0)),
            pl.BlockSpec((1, D), lambda i: (0, 0)),
            pl.BlockSpec((1, D), lambda i: (0, 0)),
            pl.BlockSpec((1, D), lambda i: (0, 0)),
            pl.BlockSpec((D, 1), lambda i: (0, 0)),
            pl.BlockSpec((1, 1), lambda i: (0, 0)),
            pl.BlockSpec((NCAND, BQ), lambda i: (0, i)),
        ],
        out_specs=[
            pl.BlockSpec((TOPK, BQ), lambda i: (0, i)),
            pl.BlockSpec((TOPK, BQ), lambda i: (0, i)),
        ],
        out_shape=[
            jax.ShapeDtypeStruct((TOPK, Q), f32),
            jax.ShapeDtypeStruct((TOPK, Q), jnp.int32),
        ],
        scratch_shapes=[pltpu.VMEM((NCAND, BQ), f32)],
    )(cemb2, query, cvec, rr_W1, rrb1r, rlngr, rlnbr, rr_W2, rb2r, cidT)

    return fsT.T, fiT.T
